# baseline (speedup 1.0000x reference)
import functools

import jax
import jax.numpy as jnp
from jax import lax
from jax.experimental import pallas as pl
from jax.experimental.pallas import tpu as pltpu
from jax.experimental.pallas import tpu_sc as plsc

N = 10000
E = 160000
D = 32
T = 16
NC, NS, L = 2, 16, 16
NW = NC * NS
C = 128
NBUF = 8
K = 40
KMAX = K
PAD_CHUNKS = NW * K
E_PAD = PAD_CHUNKS * C
NPAD = 10112
SLICE = NPAD // NS


def _tc_prep(features, B, b_ih2, b_hh2):
    def body(f_ref, b_ref, bi_ref, bh_ref, p_ref, bio_ref, bho_ref):
        p_ref[...] = jnp.dot(f_ref[...], b_ref[...],
                             preferred_element_type=jnp.float32)
        bio_ref[...] = bi_ref[...]
        bho_ref[...] = bh_ref[...]

    return pl.pallas_call(
        body,
        grid=(10,),
        in_specs=[
            pl.BlockSpec((1000, D), lambda i: (i, 0)),
            pl.BlockSpec((D, T * D), lambda i: (0, 0)),
            pl.BlockSpec((3 * D, 128), lambda i: (0, 0)),
            pl.BlockSpec((3 * D, 128), lambda i: (0, 0)),
        ],
        out_specs=[
            pl.BlockSpec((1000, T * D), lambda i: (i, 0)),
            pl.BlockSpec((3 * D, 128), lambda i: (0, 0)),
            pl.BlockSpec((3 * D, 128), lambda i: (0, 0)),
        ],
        out_shape=[
            jax.ShapeDtypeStruct((N, T * D), jnp.float32),
            jax.ShapeDtypeStruct((3 * D, 128), jnp.float32),
            jax.ShapeDtypeStruct((3 * D, 128), jnp.float32),
        ],
    )(features, B, b_ih2, b_hh2)


def _sc_agg(P_flat, srcw, typw, dstw, zeros):
    mesh = plsc.VectorSubcoreMesh(core_axis_name="c", subcore_axis_name="s")

    @functools.partial(
        pl.kernel,
        out_type=jax.ShapeDtypeStruct((NC, NPAD, D), jnp.float32),
        mesh=mesh,
        compiler_params=pltpu.CompilerParams(use_tc_tiling_on_sc=False),
        scratch_types=[
            pltpu.VMEM((KMAX, C), jnp.int32),
            pltpu.VMEM((KMAX, C), jnp.int32),
            pltpu.VMEM((KMAX, C), jnp.int32),
            pltpu.VMEM((KMAX, C), jnp.int32),
            pltpu.VMEM((NBUF, C, D), jnp.float32),
            pltpu.VMEM_SHARED((NPAD, D), jnp.float32),
            [pltpu.SemaphoreType.DMA] * NBUF,
        ],
    )
    def sc_kernel(p_hbm, src_hbm, typ_hbm, dst_hbm, z_hbm, out_hbm,
                  src_v, typ_v, idx_v, dst_v, rows_v, agg, sems):
        cid = lax.axis_index("c")
        sid = lax.axis_index("s")
        base = (sid * NC + cid) * K
        pltpu.sync_copy(z_hbm, agg.at[pl.ds(sid * SLICE, SLICE)])
        pltpu.sync_copy(src_hbm.at[pl.ds(base, KMAX)], src_v)
        pltpu.sync_copy(typ_hbm.at[pl.ds(base, KMAX)], typ_v)
        pltpu.sync_copy(dst_hbm.at[pl.ds(base, KMAX)], dst_v)
        for j in range(KMAX):
            for i in range(C // L):
                s = src_v[j, pl.ds(i * L, L)]
                t = typ_v[j, pl.ds(i * L, L)]
                idx_v[j, pl.ds(i * L, L)] = s * T + t
        plsc.subcore_barrier()

        for b in range(NBUF - 1):
            pltpu.async_copy(p_hbm.at[idx_v.at[b]], rows_v.at[b], sems[b])

        def round_body(g, carry):
            for b in range(NBUF):
                j = NBUF * g + b
                nb = (b + NBUF - 1) % NBUF

                @pl.when(j + NBUF - 1 < K)
                def _():
                    pltpu.async_copy(p_hbm.at[idx_v.at[j + NBUF - 1]],
                                     rows_v.at[nb], sems[nb])

                pltpu.make_async_copy(p_hbm.at[idx_v.at[j]],
                                      rows_v.at[b], sems[b]).wait()
                pltpu.sync_copy(rows_v.at[b], agg.at[dst_v.at[j]],
                                add=True)
            return carry

        lax.fori_loop(0, K // NBUF, round_body, 0)
        plsc.subcore_barrier()
        pltpu.sync_copy(agg.at[pl.ds(sid * SLICE, SLICE)],
                        out_hbm.at[cid, pl.ds(sid * SLICE, SLICE)])

    return sc_kernel(P_flat, srcw, typw, dstw, zeros)


def _tc_gru(aggs, features, W_ih, W_hh, b_ih2, b_hh2):
    BN = 1000

    def body(a_ref, f_ref, wi_ref, wh_ref, bi_ref, bh_ref, o_ref):
        agg = a_ref[0] + a_ref[1]
        h = f_ref[...]
        giT = lax.dot_general(wi_ref[...], agg, (((1,), (1,)), ((), ())),
                              preferred_element_type=jnp.float32)
        giT = giT + bi_ref[:, 0:1]
        ghT = lax.dot_general(wh_ref[...], h, (((1,), (1,)), ((), ())),
                              preferred_element_type=jnp.float32)
        ghT = ghT + bh_ref[:, 0:1]
        hT = h.T
        r = jax.nn.sigmoid(giT[:D] + ghT[:D])
        z = jax.nn.sigmoid(giT[D:2 * D] + ghT[D:2 * D])
        n = jnp.tanh(giT[2 * D:] + r * ghT[2 * D:])
        o_ref[...] = ((1.0 - z) * n + z * hT).T

    return pl.pallas_call(
        body,
        grid=(N // BN,),
        in_specs=[
            pl.BlockSpec((NC, BN, D), lambda i: (0, i, 0)),
            pl.BlockSpec((BN, D), lambda i: (i, 0)),
            pl.BlockSpec((3 * D, D), lambda i: (0, 0)),
            pl.BlockSpec((3 * D, D), lambda i: (0, 0)),
            pl.BlockSpec((3 * D, 128), lambda i: (0, 0)),
            pl.BlockSpec((3 * D, 128), lambda i: (0, 0)),
        ],
        out_specs=pl.BlockSpec((BN, D), lambda i: (i, 0)),
        out_shape=jax.ShapeDtypeStruct((N, D), jnp.float32),
    )(aggs, features, W_ih, W_hh, b_ih2, b_hh2)


def kernel(features, edge_index, edge_types, edge_matrix, W_ih, W_hh, b_ih, b_hh):
    B = edge_matrix.reshape(T, D, D).transpose(2, 0, 1).reshape(D, T * D)
    src = edge_index[0]
    dst = edge_index[1]
    pad = E_PAD - E
    srcw = jnp.pad(src, (0, pad)).reshape(PAD_CHUNKS, C)
    typw = jnp.pad(edge_types, (0, pad)).reshape(PAD_CHUNKS, C)
    dstw = jnp.pad(dst, (0, pad), constant_values=N).reshape(PAD_CHUNKS, C)
    zeros = jnp.zeros((SLICE, D), jnp.float32)

    b_ih2 = jnp.broadcast_to(b_ih.reshape(3 * D, 1), (3 * D, 128))
    b_hh2 = jnp.broadcast_to(b_hh.reshape(3 * D, 1), (3 * D, 128))
    P, b_ih3, b_hh3 = _tc_prep(features, B, b_ih2, b_hh2)
    aggs = _sc_agg(P.reshape(N * T, D), srcw, typw, dstw, zeros)
    return _tc_gru(aggs, features, W_ih, W_hh, b_ih3, b_hh3)


# speedup vs baseline: 1.0579x; 1.0579x over previous
import functools

import jax
import jax.numpy as jnp
from jax import lax
from jax.experimental import pallas as pl
from jax.experimental.pallas import tpu as pltpu
from jax.experimental.pallas import tpu_sc as plsc

N = 10000
E = 160000
D = 32
T = 16
NC, NS, L = 2, 16, 16
NW = NC * NS
C = 128
NBUF = 8
K = 40
KMAX = K
PAD_CHUNKS = NW * K
E_PAD = PAD_CHUNKS * C
NPAD = 10112
SLICE = NPAD // NS


def _tc_prep(features, B, b_ih2, b_hh2):
    def body(f_ref, b_ref, bi_ref, bh_ref, p_ref, bio_ref, bho_ref):
        p_ref[...] = jnp.dot(f_ref[...], b_ref[...],
                             preferred_element_type=jnp.float32)
        bio_ref[...] = bi_ref[...]
        bho_ref[...] = bh_ref[...]

    return pl.pallas_call(
        body,
        grid=(10,),
        in_specs=[
            pl.BlockSpec((1000, D), lambda i: (i, 0)),
            pl.BlockSpec((D, T * D), lambda i: (0, 0)),
            pl.BlockSpec((8, 3 * D), lambda i: (0, 0)),
            pl.BlockSpec((8, 3 * D), lambda i: (0, 0)),
        ],
        out_specs=[
            pl.BlockSpec((1000, T * D), lambda i: (i, 0)),
            pl.BlockSpec((8, 3 * D), lambda i: (0, 0)),
            pl.BlockSpec((8, 3 * D), lambda i: (0, 0)),
        ],
        out_shape=[
            jax.ShapeDtypeStruct((N, T * D), jnp.float32),
            jax.ShapeDtypeStruct((8, 3 * D), jnp.float32),
            jax.ShapeDtypeStruct((8, 3 * D), jnp.float32),
        ],
    )(features, B, b_ih2, b_hh2)


def _sc_agg(P_flat, srcw, typw, dstw, zeros):
    mesh = plsc.VectorSubcoreMesh(core_axis_name="c", subcore_axis_name="s")

    @functools.partial(
        pl.kernel,
        out_type=jax.ShapeDtypeStruct((NC, NPAD, D), jnp.float32),
        mesh=mesh,
        compiler_params=pltpu.CompilerParams(use_tc_tiling_on_sc=False),
        scratch_types=[
            pltpu.VMEM((KMAX, C), jnp.int32),
            pltpu.VMEM((KMAX, C), jnp.int32),
            pltpu.VMEM((KMAX, C), jnp.int32),
            pltpu.VMEM((KMAX, C), jnp.int32),
            pltpu.VMEM((NBUF, C, D), jnp.float32),
            pltpu.VMEM_SHARED((NPAD, D), jnp.float32),
            [pltpu.SemaphoreType.DMA] * NBUF,
        ],
    )
    def sc_kernel(p_hbm, src_hbm, typ_hbm, dst_hbm, z_hbm, out_hbm,
                  src_v, typ_v, idx_v, dst_v, rows_v, agg, sems):
        cid = lax.axis_index("c")
        sid = lax.axis_index("s")
        base = (sid * NC + cid) * K
        pltpu.sync_copy(z_hbm, agg.at[pl.ds(sid * SLICE, SLICE)])
        pltpu.sync_copy(src_hbm.at[pl.ds(base, KMAX)], src_v)
        pltpu.sync_copy(typ_hbm.at[pl.ds(base, KMAX)], typ_v)
        pltpu.sync_copy(dst_hbm.at[pl.ds(base, KMAX)], dst_v)
        for j in range(KMAX):
            for i in range(C // L):
                s = src_v[j, pl.ds(i * L, L)]
                t = typ_v[j, pl.ds(i * L, L)]
                idx_v[j, pl.ds(i * L, L)] = s * T + t
        plsc.subcore_barrier()

        for b in range(NBUF - 1):
            pltpu.async_copy(p_hbm.at[idx_v.at[b]], rows_v.at[b], sems[b])

        def round_body(g, carry):
            for b in range(NBUF):
                j = NBUF * g + b
                nb = (b + NBUF - 1) % NBUF

                @pl.when(j + NBUF - 1 < K)
                def _():
                    pltpu.async_copy(p_hbm.at[idx_v.at[j + NBUF - 1]],
                                     rows_v.at[nb], sems[nb])

                pltpu.make_async_copy(p_hbm.at[idx_v.at[j]],
                                      rows_v.at[b], sems[b]).wait()
                pltpu.sync_copy(rows_v.at[b], agg.at[dst_v.at[j]],
                                add=True)
            return carry

        lax.fori_loop(0, K // NBUF, round_body, 0)
        plsc.subcore_barrier()
        pltpu.sync_copy(agg.at[pl.ds(sid * SLICE, SLICE)],
                        out_hbm.at[cid, pl.ds(sid * SLICE, SLICE)])

    return sc_kernel(P_flat, srcw, typw, dstw, zeros)


def _tc_gru(aggs, features, W_ih, W_hh, b_ih2, b_hh2):
    BN = 1000

    def body(a_ref, f_ref, wi_ref, wh_ref, bi_ref, bh_ref, o_ref):
        agg = a_ref[0] + a_ref[1]
        h = f_ref[...]
        gi = lax.dot_general(agg, wi_ref[...], (((1,), (1,)), ((), ())),
                             preferred_element_type=jnp.float32)
        gi = gi + bi_ref[0:1, :]
        gh = lax.dot_general(h, wh_ref[...], (((1,), (1,)), ((), ())),
                             preferred_element_type=jnp.float32)
        gh = gh + bh_ref[0:1, :]
        r = jax.nn.sigmoid(gi[:, :D] + gh[:, :D])
        z = jax.nn.sigmoid(gi[:, D:2 * D] + gh[:, D:2 * D])
        n = jnp.tanh(gi[:, 2 * D:] + r * gh[:, 2 * D:])
        o_ref[...] = (1.0 - z) * n + z * h

    return pl.pallas_call(
        body,
        grid=(N // BN,),
        in_specs=[
            pl.BlockSpec((NC, BN, D), lambda i: (0, i, 0)),
            pl.BlockSpec((BN, D), lambda i: (i, 0)),
            pl.BlockSpec((3 * D, D), lambda i: (0, 0)),
            pl.BlockSpec((3 * D, D), lambda i: (0, 0)),
            pl.BlockSpec((8, 3 * D), lambda i: (0, 0)),
            pl.BlockSpec((8, 3 * D), lambda i: (0, 0)),
        ],
        out_specs=pl.BlockSpec((BN, D), lambda i: (i, 0)),
        out_shape=jax.ShapeDtypeStruct((N, D), jnp.float32),
    )(aggs, features, W_ih, W_hh, b_ih2, b_hh2)


def kernel(features, edge_index, edge_types, edge_matrix, W_ih, W_hh, b_ih, b_hh):
    B = edge_matrix.reshape(T, D, D).transpose(2, 0, 1).reshape(D, T * D)
    src = edge_index[0]
    dst = edge_index[1]
    pad = E_PAD - E
    srcw = jnp.pad(src, (0, pad)).reshape(PAD_CHUNKS, C)
    typw = jnp.pad(edge_types, (0, pad)).reshape(PAD_CHUNKS, C)
    dstw = jnp.pad(dst, (0, pad), constant_values=N).reshape(PAD_CHUNKS, C)
    zeros = jnp.zeros((SLICE, D), jnp.float32)

    b_ih2 = jnp.broadcast_to(b_ih.reshape(1, 3 * D), (8, 3 * D))
    b_hh2 = jnp.broadcast_to(b_hh.reshape(1, 3 * D), (8, 3 * D))
    P, b_ih3, b_hh3 = _tc_prep(features, B, b_ih2, b_hh2)
    aggs = _sc_agg(P.reshape(N * T, D), srcw, typw, dstw, zeros)
    return _tc_gru(aggs, features, W_ih, W_hh, b_ih3, b_hh3)
